# baseline (speedup 1.0000x reference)
import functools

import jax
import jax.numpy as jnp
from jax import lax
from jax.experimental import pallas as pl
from jax.experimental.pallas import tpu as pltpu
from jax.experimental.pallas import tpu_sc as plsc

_HEADS = 12
_NB = 32
_SCALE = 0.125
_BI = 512
_CH = 256
_R = _BI // _CH
_C0 = _BI + 512
_DW = _C0 + _BI + 512
_DWP = 2560
_NW = 32
_XPW = _DWP // _NW
_THRESH = (8, 12, 16, 23, 32, 46, 64, 91)


def _bucket_from_rel(rel):
    na = jnp.abs(rel)
    zero = jnp.zeros(rel.shape, jnp.int32)
    one = jnp.full(rel.shape, 1, jnp.int32)
    vil = jnp.full(rel.shape, 7, jnp.int32)
    for t in _THRESH:
        vil = vil + jnp.where(na >= t, one, zero)
    bk = jnp.where(na < 8, na, vil)
    return bk + jnp.where(rel > 0, jnp.full(rel.shape, 16, jnp.int32), zero)


def _sc_diag_body(tbl_hbm, out_hbm, tbl_v, chunk_v):
    wid = lax.axis_index("s") * 2 + lax.axis_index("c")
    base = wid * _XPW
    pltpu.sync_copy(tbl_hbm, tbl_v)
    lanes = lax.iota(jnp.int32, 16)
    for h in range(_HEADS):
        for v in range(_XPW // 16):
            x = lanes + (base + 16 * v)
            rel = x - (_C0 + 7)
            bk = _bucket_from_rel(rel)
            idx = bk * _HEADS + h
            vals = plsc.load_gather(tbl_v, [idx])
            chunk_v[pl.ds(16 * v, 16)] = vals * _SCALE
        pltpu.sync_copy(chunk_v, out_hbm.at[pl.ds(h * _DWP + base, _XPW)])


_sc_diag = functools.partial(
    pl.kernel,
    out_type=jax.ShapeDtypeStruct((_HEADS * _DWP,), jnp.float32),
    mesh=plsc.VectorSubcoreMesh(core_axis_name="c", subcore_axis_name="s"),
    scratch_types=[
        pltpu.VMEM((_NB * _HEADS,), jnp.float32),
        pltpu.VMEM((_XPW,), jnp.float32),
    ],
    compiler_params=pltpu.CompilerParams(needs_layout_passes=False),
)(_sc_diag_body)


def _tc_body(qk_ref, tbl_ref, diag_ref, out_ref, d8_ref):
    h = pl.program_id(0)
    ib = pl.program_id(1)
    ncols = out_ref.shape[-1]
    nchunks = ncols // _CH

    @pl.when(ib == 0)
    def _build_d8():
        for s in range(8):
            d8_ref[s:s + 1, :] = diag_ref[pl.ds(h, 1), 7 - s:7 - s + _DW]

    c_lo = tbl_ref[15, h] * _SCALE
    c_up = tbl_ref[31, h] * _SCALE

    n_ib = pl.num_programs(1)
    for ibv in range(n_ib):

        @pl.when(ib == ibv)
        def _block(ibv=ibv):
            for cc in range(nchunks):
                dd = cc - _R * ibv
                if dd < -1:
                    out_ref[0, 0, :, _CH * cc:_CH * (cc + 1)] = (
                        qk_ref[0, 0, :, _CH * cc:_CH * (cc + 1)] + c_lo
                    )
                elif dd > _R:
                    out_ref[0, 0, :, _CH * cc:_CH * (cc + 1)] = (
                        qk_ref[0, 0, :, _CH * cc:_CH * (cc + 1)] + c_up
                    )
                else:
                    for q in range(_BI // 8):
                        off = _CH * dd + _C0 - 8 * q
                        out_ref[0, 0, 8 * q:8 * q + 8,
                                _CH * cc:_CH * (cc + 1)] = (
                            qk_ref[0, 0, 8 * q:8 * q + 8,
                                   _CH * cc:_CH * (cc + 1)]
                            + d8_ref[:, off:off + _CH]
                        )


@jax.jit
def kernel(qk_dots, rel_bias_table):
    i = qk_dots.shape[-2]
    j = qk_dots.shape[-1]
    diag = _sc_diag(rel_bias_table.reshape(-1)).reshape(_HEADS, _DWP)
    grid = (_HEADS, i // _BI)
    return pl.pallas_call(
        _tc_body,
        grid=grid,
        in_specs=[
            pl.BlockSpec((1, 1, _BI, j), lambda h, ib: (0, h, ib, 0)),
            pl.BlockSpec(memory_space=pltpu.SMEM),
            pl.BlockSpec((_HEADS, _DWP), lambda h, ib: (0, 0)),
        ],
        out_specs=pl.BlockSpec((1, 1, _BI, j), lambda h, ib: (0, h, ib, 0)),
        out_shape=jax.ShapeDtypeStruct(qk_dots.shape, qk_dots.dtype),
        scratch_shapes=[pltpu.VMEM((8, _DW), jnp.float32)],
    )(qk_dots, rel_bias_table, diag)


# speedup vs baseline: 1.0104x; 1.0104x over previous
import functools

import jax
import jax.numpy as jnp
from jax import lax
from jax.experimental import pallas as pl
from jax.experimental.pallas import tpu as pltpu
from jax.experimental.pallas import tpu_sc as plsc

_HEADS = 12
_NB = 32
_SCALE = 0.125
_BI = 1024
_CH = 256
_R = _BI // _CH
_C0 = _BI + 512
_DW = _C0 + _BI + 512
_DWP = 3584
_NW = 32
_XPW = _DWP // _NW
_THRESH = (8, 12, 16, 23, 32, 46, 64, 91)


def _bucket_from_rel(rel):
    na = jnp.abs(rel)
    zero = jnp.zeros(rel.shape, jnp.int32)
    one = jnp.full(rel.shape, 1, jnp.int32)
    vil = jnp.full(rel.shape, 7, jnp.int32)
    for t in _THRESH:
        vil = vil + jnp.where(na >= t, one, zero)
    bk = jnp.where(na < 8, na, vil)
    return bk + jnp.where(rel > 0, jnp.full(rel.shape, 16, jnp.int32), zero)


def _sc_diag_body(tbl_hbm, out_hbm, tbl_v, chunk_v):
    wid = lax.axis_index("s") * 2 + lax.axis_index("c")
    base = wid * _XPW
    pltpu.sync_copy(tbl_hbm, tbl_v)
    lanes = lax.iota(jnp.int32, 16)
    for h in range(_HEADS):
        for v in range(_XPW // 16):
            x = lanes + (base + 16 * v)
            rel = x - (_C0 + 7)
            bk = _bucket_from_rel(rel)
            idx = bk * _HEADS + h
            vals = plsc.load_gather(tbl_v, [idx])
            chunk_v[pl.ds(16 * v, 16)] = vals * _SCALE
        pltpu.sync_copy(chunk_v, out_hbm.at[pl.ds(h * _DWP + base, _XPW)])


_sc_diag = functools.partial(
    pl.kernel,
    out_type=jax.ShapeDtypeStruct((_HEADS * _DWP,), jnp.float32),
    mesh=plsc.VectorSubcoreMesh(core_axis_name="c", subcore_axis_name="s"),
    scratch_types=[
        pltpu.VMEM((_NB * _HEADS,), jnp.float32),
        pltpu.VMEM((_XPW,), jnp.float32),
    ],
    compiler_params=pltpu.CompilerParams(needs_layout_passes=False),
)(_sc_diag_body)


def _tc_body(qk_ref, tbl_ref, diag_ref, out_ref, d8_ref):
    h = pl.program_id(0)
    ib = pl.program_id(1)
    ncols = out_ref.shape[-1]
    nchunks = ncols // _CH

    @pl.when(ib == 0)
    def _build_d8():
        for s in range(8):
            d8_ref[s:s + 1, :] = diag_ref[pl.ds(h, 1), 7 - s:7 - s + _DW]

    c_lo = tbl_ref[15, h] * _SCALE
    c_up = tbl_ref[31, h] * _SCALE

    n_ib = pl.num_programs(1)
    for ibv in range(n_ib):

        @pl.when(ib == ibv)
        def _block(ibv=ibv):
            for cc in range(nchunks):
                dd = cc - _R * ibv
                if dd < -1:
                    out_ref[0, 0, :, _CH * cc:_CH * (cc + 1)] = (
                        qk_ref[0, 0, :, _CH * cc:_CH * (cc + 1)] + c_lo
                    )
                elif dd > _R:
                    out_ref[0, 0, :, _CH * cc:_CH * (cc + 1)] = (
                        qk_ref[0, 0, :, _CH * cc:_CH * (cc + 1)] + c_up
                    )
                else:
                    for q in range(_BI // 8):
                        off = _CH * dd + _C0 - 8 * q
                        out_ref[0, 0, 8 * q:8 * q + 8,
                                _CH * cc:_CH * (cc + 1)] = (
                            qk_ref[0, 0, 8 * q:8 * q + 8,
                                   _CH * cc:_CH * (cc + 1)]
                            + d8_ref[:, off:off + _CH]
                        )


@jax.jit
def kernel(qk_dots, rel_bias_table):
    i = qk_dots.shape[-2]
    j = qk_dots.shape[-1]
    diag = _sc_diag(rel_bias_table.reshape(-1)).reshape(_HEADS, _DWP)
    grid = (_HEADS, i // _BI)
    return pl.pallas_call(
        _tc_body,
        grid=grid,
        in_specs=[
            pl.BlockSpec((1, 1, _BI, j), lambda h, ib: (0, h, ib, 0)),
            pl.BlockSpec(memory_space=pltpu.SMEM),
            pl.BlockSpec((_HEADS, _DWP), lambda h, ib: (0, 0)),
        ],
        out_specs=pl.BlockSpec((1, 1, _BI, j), lambda h, ib: (0, h, ib, 0)),
        out_shape=jax.ShapeDtypeStruct(qk_dots.shape, qk_dots.dtype),
        scratch_shapes=[pltpu.VMEM((8, _DW), jnp.float32)],
    )(qk_dots, rel_bias_table, diag)
